# baseline (device time: 1578352 ns/iter reference)
import jax
import jax.numpy as jnp
from jax import lax
from jax.experimental import pallas as pl
from jax.experimental.pallas import tpu as pltpu

M = 4096
N = 8192
K = 4096
HALF = 2048

BM, BN, BK = 512, 1024, 512


def _mm_body(x_ref, dy_ref, o_ref, acc_ref):
    @pl.when(pl.program_id(2) == 0)
    def _():
        acc_ref[...] = jnp.zeros_like(acc_ref)

    acc_ref[...] += lax.dot_general(
        x_ref[...],
        dy_ref[...],
        dimension_numbers=(((0,), (0,)), ((), ())),
        preferred_element_type=jnp.float32,
    )

    @pl.when(pl.program_id(2) == pl.num_programs(2) - 1)
    def _():
        o_ref[...] = acc_ref[...]


def _partial_matmul(x, dy):
    return pl.pallas_call(
        _mm_body,
        grid=(M // BM, N // BN, K // BK),
        in_specs=[
            pl.BlockSpec((BK, BM), lambda i, j, k: (k, i)),
            pl.BlockSpec((BK, BN), lambda i, j, k: (k, j)),
        ],
        out_specs=pl.BlockSpec((BM, BN), lambda i, j, k: (i, j)),
        out_shape=jax.ShapeDtypeStruct((M, N), jnp.float32),
        scratch_shapes=[pltpu.VMEM((BM, BN), jnp.float32)],
        compiler_params=pltpu.CompilerParams(
            dimension_semantics=("parallel", "parallel", "arbitrary"),
        ),
    )(x, dy)


def _exchange_body(p_ref, r_ref, send_sem, recv_sem):
    ix = lax.axis_index("x")
    iy = lax.axis_index("y")
    iz = lax.axis_index("z")
    nbr = (ix, 1 - iy, iz)

    barrier = pltpu.get_barrier_semaphore()
    pl.semaphore_signal(
        barrier, inc=1, device_id=nbr, device_id_type=pl.DeviceIdType.MESH
    )
    pl.semaphore_wait(barrier, 1)

    rdma = pltpu.make_async_remote_copy(
        src_ref=p_ref.at[pl.ds((1 - iy) * HALF, HALF)],
        dst_ref=r_ref,
        send_sem=send_sem,
        recv_sem=recv_sem,
        device_id=nbr,
        device_id_type=pl.DeviceIdType.MESH,
    )
    rdma.start()
    rdma.wait()


def _exchange(p):
    return pl.pallas_call(
        _exchange_body,
        in_specs=[pl.BlockSpec(memory_space=pl.ANY)],
        out_specs=pl.BlockSpec(memory_space=pl.ANY),
        out_shape=jax.ShapeDtypeStruct((HALF, N), jnp.float32),
        scratch_shapes=[pltpu.SemaphoreType.DMA, pltpu.SemaphoreType.DMA],
        compiler_params=pltpu.CompilerParams(collective_id=0),
    )(p)


def _add_body(iy_ref, p_ref, r_ref, o_ref):
    del iy_ref
    o_ref[...] = p_ref[...] + r_ref[...]


def _add(iy, p, r):
    return pl.pallas_call(
        _add_body,
        grid_spec=pltpu.PrefetchScalarGridSpec(
            num_scalar_prefetch=1,
            grid=(HALF // BM, N // BN),
            in_specs=[
                pl.BlockSpec(
                    (BM, BN),
                    lambda i, j, iy_ref: (iy_ref[0] * (HALF // BM) + i, j),
                ),
                pl.BlockSpec((BM, BN), lambda i, j, iy_ref: (i, j)),
            ],
            out_specs=pl.BlockSpec((BM, BN), lambda i, j, iy_ref: (i, j)),
        ),
        out_shape=jax.ShapeDtypeStruct((HALF, N), jnp.float32),
    )(iy, p, r)


def kernel(x, dy):
    p = _partial_matmul(x, dy)
    r = _exchange(p)
    iy = jnp.full((1,), lax.axis_index("y"), jnp.int32)
    return _add(iy, p, r)


# device time: 707875 ns/iter; 2.2297x vs baseline; 2.2297x over previous
import jax
import jax.numpy as jnp
from jax import lax
from jax.experimental import pallas as pl
from jax.experimental.pallas import tpu as pltpu

M = 4096
N = 8192
K = 4096
HALF = 2048
BLK = 1024

BM, BN, BK = 512, 1024, 512

C = 8
CH = BLK // C

_MESH = pl.DeviceIdType.MESH


def _mm_body(b_ref, x_ref, dy_ref, o_ref, acc_ref):
    del b_ref

    @pl.when(pl.program_id(2) == 0)
    def _():
        acc_ref[...] = jnp.zeros_like(acc_ref)

    acc_ref[...] += lax.dot_general(
        x_ref[...],
        dy_ref[...],
        dimension_numbers=(((0,), (0,)), ((), ())),
        preferred_element_type=jnp.float32,
    )

    @pl.when(pl.program_id(2) == pl.num_programs(2) - 1)
    def _():
        o_ref[...] = acc_ref[...]


def _block_matmul(b, x, dy):
    return pl.pallas_call(
        _mm_body,
        grid_spec=pltpu.PrefetchScalarGridSpec(
            num_scalar_prefetch=1,
            grid=(BLK // BM, N // BN, K // BK),
            in_specs=[
                pl.BlockSpec(
                    (BK, BM),
                    lambda i, j, k, b: (k, b[0] * (BLK // BM) + i),
                ),
                pl.BlockSpec((BK, BN), lambda i, j, k, b: (k, j)),
            ],
            out_specs=pl.BlockSpec((BM, BN), lambda i, j, k, b: (i, j)),
            scratch_shapes=[pltpu.VMEM((BM, BN), jnp.float32)],
        ),
        out_shape=jax.ShapeDtypeStruct((BLK, N), jnp.float32),
        compiler_params=pltpu.CompilerParams(
            dimension_semantics=("parallel", "parallel", "arbitrary"),
        ),
    )(b, x, dy)


def _comm_body(
    p_ref,
    o_ref,
    yb_ref,
    va_ref,
    vb_ref,
    vf_ref,
    l0,
    l1,
    l2,
    ysend_sems,
    yrecv_sems,
    xsend_sems,
    xrecv_sems,
    zsend_sems,
    zrecv_sems,
):
    ix = lax.axis_index("x")
    iy = lax.axis_index("y")
    iz = lax.axis_index("z")
    y_nbr = (ix, 1 - iy, iz)
    x_nbr = (1 - ix, iy, iz)
    z_nbr = (ix, iy, 1 - iz)
    is_red = ix == iy

    my_off = iz * BLK
    ot_off = (1 - iz) * BLK

    barrier = pltpu.get_barrier_semaphore()
    for nbr in (x_nbr, y_nbr, z_nbr):
        pl.semaphore_signal(barrier, inc=1, device_id=nbr, device_id_type=_MESH)
    pl.semaphore_wait(barrier, 3)

    def y_edge(c):
        sl = pl.ds(c * CH, CH)
        return pltpu.make_async_remote_copy(
            src_ref=p_ref.at[sl],
            dst_ref=yb_ref.at[sl],
            send_sem=ysend_sems.at[c],
            recv_sem=yrecv_sems.at[c],
            device_id=y_nbr,
            device_id_type=_MESH,
        )

    def x_edge(c):
        sl = pl.ds(my_off + c * CH, CH)
        return pltpu.make_async_remote_copy(
            src_ref=o_ref.at[sl],
            dst_ref=o_ref.at[sl],
            send_sem=xsend_sems.at[c],
            recv_sem=xrecv_sems.at[c],
            device_id=x_nbr,
            device_id_type=_MESH,
        )

    def z_out_edge(c):
        sl = pl.ds(my_off + c * CH, CH)
        return pltpu.make_async_remote_copy(
            src_ref=o_ref.at[sl],
            dst_ref=o_ref.at[sl],
            send_sem=zsend_sems.at[c],
            recv_sem=zrecv_sems.at[c],
            device_id=z_nbr,
            device_id_type=_MESH,
        )

    def z_in_edge(c):
        sl = pl.ds(ot_off + c * CH, CH)
        return pltpu.make_async_remote_copy(
            src_ref=o_ref.at[sl],
            dst_ref=o_ref.at[sl],
            send_sem=zsend_sems.at[c],
            recv_sem=zrecv_sems.at[c],
            device_id=z_nbr,
            device_id_type=_MESH,
        )

    @pl.when(is_red)
    def _():
        sends = []
        for c in range(C):
            sl = pl.ds(c * CH, CH)
            y_edge(c).wait_recv()
            cp_a = pltpu.make_async_copy(yb_ref.at[sl], va_ref, l0)
            cp_a.start()
            cp_b = pltpu.make_async_copy(p_ref.at[sl], vb_ref, l1)
            cp_b.start()
            cp_a.wait()
            cp_b.wait()
            vf_ref[...] = va_ref[...] + vb_ref[...]
            cp_o = pltpu.make_async_copy(
                vf_ref, o_ref.at[pl.ds(my_off + c * CH, CH)], l2
            )
            cp_o.start()
            cp_o.wait()
            rx = x_edge(c)
            rx.start()
            rz = z_out_edge(c)
            rz.start()
            sends.append((rx, rz))
        for c in range(C):
            z_in_edge(c).wait_recv()
        for rx, rz in sends:
            rx.wait_send()
            rz.wait_send()

    @pl.when(jnp.logical_not(is_red))
    def _():
        y_sends = []
        for c in range(C):
            ry = y_edge(c)
            ry.start()
            y_sends.append(ry)
        z_sends = []
        for c in range(C):
            x_edge(c).wait_recv()
            rz = z_out_edge(c)
            rz.start()
            z_sends.append(rz)
        for c in range(C):
            z_in_edge(c).wait_recv()
        for ry in y_sends:
            ry.wait_send()
        for rz in z_sends:
            rz.wait_send()


def _comm(p):
    return pl.pallas_call(
        _comm_body,
        in_specs=[pl.BlockSpec(memory_space=pl.ANY)],
        out_specs=[
            pl.BlockSpec(memory_space=pl.ANY),
            pl.BlockSpec(memory_space=pl.ANY),
        ],
        out_shape=[
            jax.ShapeDtypeStruct((HALF, N), jnp.float32),
            jax.ShapeDtypeStruct((BLK, N), jnp.float32),
        ],
        scratch_shapes=[
            pltpu.VMEM((CH, N), jnp.float32),
            pltpu.VMEM((CH, N), jnp.float32),
            pltpu.VMEM((CH, N), jnp.float32),
            pltpu.SemaphoreType.DMA,
            pltpu.SemaphoreType.DMA,
            pltpu.SemaphoreType.DMA,
            pltpu.SemaphoreType.DMA((C,)),
            pltpu.SemaphoreType.DMA((C,)),
            pltpu.SemaphoreType.DMA((C,)),
            pltpu.SemaphoreType.DMA((C,)),
            pltpu.SemaphoreType.DMA((C,)),
            pltpu.SemaphoreType.DMA((C,)),
        ],
        compiler_params=pltpu.CompilerParams(collective_id=0),
    )(p)


def kernel(x, dy):
    ix = lax.axis_index("x")
    iz = lax.axis_index("z")
    b = jnp.full((1,), 2 * ix + iz, jnp.int32)
    p = _block_matmul(b, x, dy)
    out, _ = _comm(p)
    return out


# device time: 575823 ns/iter; 2.7410x vs baseline; 1.2293x over previous
import jax
import jax.numpy as jnp
from jax import lax
from jax.experimental import pallas as pl
from jax.experimental.pallas import tpu as pltpu

M = 4096
N = 8192
K = 4096
HALF = 2048
HB = 512

BM, BN, BK = 512, 1024, 512

C = 4
CH = HB // C

_MESH = pl.DeviceIdType.MESH


def _mm_body(s_ref, x_ref, dy_ref, o_ref, acc_ref):
    del s_ref

    @pl.when(pl.program_id(2) == 0)
    def _():
        acc_ref[...] = jnp.zeros_like(acc_ref)

    acc_ref[...] += lax.dot_general(
        x_ref[...],
        dy_ref[...],
        dimension_numbers=(((0,), (0,)), ((), ())),
        preferred_element_type=jnp.float32,
    )

    @pl.when(pl.program_id(2) == pl.num_programs(2) - 1)
    def _():
        o_ref[...] = acc_ref[...]


def _half_matmul(s, x, dy):
    return pl.pallas_call(
        _mm_body,
        grid_spec=pltpu.PrefetchScalarGridSpec(
            num_scalar_prefetch=1,
            grid=(2, N // BN, K // BK),
            in_specs=[
                pl.BlockSpec((BK, HB), lambda i, j, k, s: (k, s[i])),
                pl.BlockSpec((BK, BN), lambda i, j, k, s: (k, j)),
            ],
            out_specs=pl.BlockSpec((HB, BN), lambda i, j, k, s: (i, j)),
            scratch_shapes=[pltpu.VMEM((HB, BN), jnp.float32)],
        ),
        out_shape=jax.ShapeDtypeStruct((2 * HB, N), jnp.float32),
        compiler_params=pltpu.CompilerParams(
            dimension_semantics=("parallel", "parallel", "arbitrary"),
        ),
    )(s, x, dy)


def _comm_body(
    p_ref,
    o_ref,
    yb_ref,
    va_ref,
    vb_ref,
    vf_ref,
    l0,
    l1,
    l2,
    ya_s,
    ya_r,
    xp_s,
    xp_r,
    zp_s,
    zp_r,
    xr_s,
    xr_r,
    zr_s,
    zr_r,
):
    ix = lax.axis_index("x")
    iy = lax.axis_index("y")
    iz = lax.axis_index("z")
    y_nbr = (ix, 1 - iy, iz)
    x_nbr = (1 - ix, iy, iz)
    z_nbr = (ix, iy, 1 - iz)

    q = 2 * ix + iz
    q_x = 2 * (1 - ix) + iz
    q_z = 2 * ix + (1 - iz)
    q_d = 2 * (1 - ix) + (1 - iz)
    off = q * HB
    off_x = q_x * HB
    off_z = q_z * HB
    off_d = q_d * HB

    barrier = pltpu.get_barrier_semaphore()
    for nbr in (x_nbr, y_nbr, z_nbr):
        pl.semaphore_signal(barrier, inc=1, device_id=nbr, device_id_type=_MESH)
    pl.semaphore_wait(barrier, 3)

    def y_edge(c):
        return pltpu.make_async_remote_copy(
            src_ref=p_ref.at[pl.ds(HB + c * CH, CH)],
            dst_ref=yb_ref.at[pl.ds(c * CH, CH)],
            send_sem=ya_s.at[c],
            recv_sem=ya_r.at[c],
            device_id=y_nbr,
            device_id_type=_MESH,
        )

    def o_edge(row_off, send, recv, c, dev):
        sl = pl.ds(row_off + c * CH, CH)
        return pltpu.make_async_remote_copy(
            src_ref=o_ref.at[sl],
            dst_ref=o_ref.at[sl],
            send_sem=send.at[c],
            recv_sem=recv.at[c],
            device_id=dev,
            device_id_type=_MESH,
        )

    y_sends = []
    for c in range(C):
        r = y_edge(c)
        r.start()
        y_sends.append(r)

    prim_sends = []
    for c in range(C):
        sl = pl.ds(c * CH, CH)
        y_edge(c).wait_recv()
        cp_a = pltpu.make_async_copy(yb_ref.at[sl], va_ref, l0)
        cp_a.start()
        cp_b = pltpu.make_async_copy(p_ref.at[sl], vb_ref, l1)
        cp_b.start()
        cp_a.wait()
        cp_b.wait()
        vf_ref[...] = va_ref[...] + vb_ref[...]
        cp_o = pltpu.make_async_copy(vf_ref, o_ref.at[pl.ds(off + c * CH, CH)], l2)
        cp_o.start()
        cp_o.wait()
        rx = o_edge(off, xp_s, xp_r, c, x_nbr)
        rx.start()
        rz = o_edge(off, zp_s, zp_r, c, z_nbr)
        rz.start()
        prim_sends += [rx, rz]

    relay_sends = []
    for c in range(C // 2):
        o_edge(off_x, xp_s, xp_r, c, x_nbr).wait_recv()
        r = o_edge(off_x, zr_s, zr_r, c, z_nbr)
        r.start()
        relay_sends.append(r)
    for c in range(C // 2, C):
        o_edge(off_z, zp_s, zp_r, c, z_nbr).wait_recv()
        r = o_edge(off_z, xr_s, xr_r, c, x_nbr)
        r.start()
        relay_sends.append(r)

    for c in range(C // 2, C):
        o_edge(off_x, xp_s, xp_r, c, x_nbr).wait_recv()
    for c in range(C // 2):
        o_edge(off_z, zp_s, zp_r, c, z_nbr).wait_recv()
    for c in range(C // 2):
        o_edge(off_d, zr_s, zr_r, c, z_nbr).wait_recv()
    for c in range(C // 2, C):
        o_edge(off_d, xr_s, xr_r, c, x_nbr).wait_recv()

    for r in y_sends + prim_sends + relay_sends:
        r.wait_send()


def _comm(p):
    return pl.pallas_call(
        _comm_body,
        in_specs=[pl.BlockSpec(memory_space=pl.ANY)],
        out_specs=[
            pl.BlockSpec(memory_space=pl.ANY),
            pl.BlockSpec(memory_space=pl.ANY),
        ],
        out_shape=[
            jax.ShapeDtypeStruct((HALF, N), jnp.float32),
            jax.ShapeDtypeStruct((HB, N), jnp.float32),
        ],
        scratch_shapes=[
            pltpu.VMEM((CH, N), jnp.float32),
            pltpu.VMEM((CH, N), jnp.float32),
            pltpu.VMEM((CH, N), jnp.float32),
            pltpu.SemaphoreType.DMA,
            pltpu.SemaphoreType.DMA,
            pltpu.SemaphoreType.DMA,
            pltpu.SemaphoreType.DMA((C,)),
            pltpu.SemaphoreType.DMA((C,)),
            pltpu.SemaphoreType.DMA((C,)),
            pltpu.SemaphoreType.DMA((C,)),
            pltpu.SemaphoreType.DMA((C,)),
            pltpu.SemaphoreType.DMA((C,)),
            pltpu.SemaphoreType.DMA((C,)),
            pltpu.SemaphoreType.DMA((C,)),
            pltpu.SemaphoreType.DMA((C,)),
            pltpu.SemaphoreType.DMA((C,)),
        ],
        compiler_params=pltpu.CompilerParams(collective_id=0),
    )(p)


def kernel(x, dy):
    ix = lax.axis_index("x")
    iy = lax.axis_index("y")
    iz = lax.axis_index("z")
    q = 2 * ix + iz
    h1 = 4 * iy + q
    h2 = 4 * (1 - iy) + q
    s = jnp.stack([h1, h2]).astype(jnp.int32)
    p = _half_matmul(s, x, dy)
    out, _ = _comm(p)
    return out


# device time: 560527 ns/iter; 2.8158x vs baseline; 1.0273x over previous
import jax
import jax.numpy as jnp
from jax import lax
from jax.experimental import pallas as pl
from jax.experimental.pallas import tpu as pltpu

M = 4096
N = 8192
K = 4096
HALF = 2048
HB = 512

C = 4
CH = HB // C
T = 2 * C
BK = 256
NK = K // BK

_MESH = pl.DeviceIdType.MESH


def _fused_body(
    s_ref,
    x_ref,
    dy_ref,
    o_ref,
    yb_ref,
    acc_c,
    acc_o,
    vb_ref,
    l1,
    l2,
    ya_s,
    ya_r,
    xp_s,
    xp_r,
    zp_s,
    zp_r,
    xr_s,
    xr_r,
    zr_s,
    zr_r,
):
    del s_ref
    t = pl.program_id(0)
    k = pl.program_id(1)
    j = t // 2
    is_cross = (t % 2) == 0

    ix = lax.axis_index("x")
    iy = lax.axis_index("y")
    iz = lax.axis_index("z")
    y_nbr = (ix, 1 - iy, iz)
    x_nbr = (1 - ix, iy, iz)
    z_nbr = (ix, iy, 1 - iz)

    q = 2 * ix + iz
    off = q * HB
    off_x = (2 * (1 - ix) + iz) * HB
    off_z = (2 * ix + (1 - iz)) * HB
    off_d = (2 * (1 - ix) + (1 - iz)) * HB

    def y_edge(c):
        return pltpu.make_async_remote_copy(
            src_ref=acc_c.at[c % 2],
            dst_ref=yb_ref.at[pl.ds(c * CH, CH)],
            send_sem=ya_s.at[c],
            recv_sem=ya_r.at[c],
            device_id=y_nbr,
            device_id_type=_MESH,
        )

    def prim_edge(c, send, recv, dev):
        return pltpu.make_async_remote_copy(
            src_ref=acc_o.at[c % 2],
            dst_ref=o_ref.at[pl.ds(off + c * CH, CH)],
            send_sem=send.at[c],
            recv_sem=recv.at[c],
            device_id=dev,
            device_id_type=_MESH,
        )

    def o_edge(row_off, send, recv, c, dev):
        sl = pl.ds(row_off + c * CH, CH)
        return pltpu.make_async_remote_copy(
            src_ref=o_ref.at[sl],
            dst_ref=o_ref.at[sl],
            send_sem=send.at[c],
            recv_sem=recv.at[c],
            device_id=dev,
            device_id_type=_MESH,
        )

    def local_store(c):
        return pltpu.make_async_copy(
            acc_o.at[c % 2], o_ref.at[pl.ds(off + c * CH, CH)], l2.at[c]
        )

    @pl.when((t == 0) & (k == 0))
    def _():
        barrier = pltpu.get_barrier_semaphore()
        for nbr in (x_nbr, y_nbr, z_nbr):
            pl.semaphore_signal(
                barrier, inc=1, device_id=nbr, device_id_type=_MESH
            )
        pl.semaphore_wait(barrier, 3)

    prod = lax.dot_general(
        x_ref[...],
        dy_ref[...],
        dimension_numbers=(((0,), (0,)), ((), ())),
        preferred_element_type=jnp.float32,
    )

    sl = j % 2

    @pl.when(is_cross)
    def _():
        @pl.when(k == 0)
        def _():
            @pl.when(j >= 2)
            def _():
                y_edge(j - 2).wait_send()

            acc_c[sl, :, :] = prod

        @pl.when(k != 0)
        def _():
            acc_c[sl, :, :] += prod

    @pl.when(jnp.logical_not(is_cross))
    def _():
        @pl.when(k == 0)
        def _():
            @pl.when(j >= 2)
            def _():
                prim_edge(j - 2, xp_s, xp_r, x_nbr).wait_send()
                prim_edge(j - 2, zp_s, zp_r, z_nbr).wait_send()
                local_store(j - 2).wait()

            acc_o[sl, :, :] = prod

        @pl.when(k != 0)
        def _():
            acc_o[sl, :, :] += prod

    @pl.when((k == NK - 1) & is_cross)
    def _():
        y_edge(j).start()

    @pl.when((k == NK - 1) & jnp.logical_not(is_cross))
    def _():
        y_edge(j).wait_recv()
        cp = pltpu.make_async_copy(yb_ref.at[pl.ds(j * CH, CH)], vb_ref, l1)
        cp.start()
        cp.wait()
        acc_o[sl, :, :] += vb_ref[...]
        local_store(j).start()
        prim_edge(j, xp_s, xp_r, x_nbr).start()
        prim_edge(j, zp_s, zp_r, z_nbr).start()

    @pl.when((t == T - 1) & (k == NK - 1))
    def _():
        relays = []
        for c in range(C // 2):
            o_edge(off_x, xp_s, xp_r, c, x_nbr).wait_recv()
            r = o_edge(off_x, zr_s, zr_r, c, z_nbr)
            r.start()
            relays.append(r)
        for c in range(C // 2, C):
            o_edge(off_z, zp_s, zp_r, c, z_nbr).wait_recv()
            r = o_edge(off_z, xr_s, xr_r, c, x_nbr)
            r.start()
            relays.append(r)
        for c in range(C // 2, C):
            o_edge(off_x, xp_s, xp_r, c, x_nbr).wait_recv()
        for c in range(C // 2):
            o_edge(off_z, zp_s, zp_r, c, z_nbr).wait_recv()
        for c in range(C // 2):
            o_edge(off_d, zr_s, zr_r, c, z_nbr).wait_recv()
        for c in range(C // 2, C):
            o_edge(off_d, xr_s, xr_r, c, x_nbr).wait_recv()
        for c in range(C - 2, C):
            y_edge(c).wait_send()
            prim_edge(c, xp_s, xp_r, x_nbr).wait_send()
            prim_edge(c, zp_s, zp_r, z_nbr).wait_send()
            local_store(c).wait()
        for r in relays:
            r.wait_send()


def _fused(s, x, dy):
    return pl.pallas_call(
        _fused_body,
        grid_spec=pltpu.PrefetchScalarGridSpec(
            num_scalar_prefetch=1,
            grid=(T, NK),
            in_specs=[
                pl.BlockSpec((BK, CH), lambda t, k, s: (k, s[t])),
                pl.BlockSpec((BK, N), lambda t, k, s: (k, 0)),
            ],
            out_specs=[
                pl.BlockSpec(memory_space=pl.ANY),
                pl.BlockSpec(memory_space=pl.ANY),
            ],
            scratch_shapes=[
                pltpu.VMEM((2, CH, N), jnp.float32),
                pltpu.VMEM((2, CH, N), jnp.float32),
                pltpu.VMEM((CH, N), jnp.float32),
                pltpu.SemaphoreType.DMA,
                pltpu.SemaphoreType.DMA((C,)),
                pltpu.SemaphoreType.DMA((C,)),
                pltpu.SemaphoreType.DMA((C,)),
                pltpu.SemaphoreType.DMA((C,)),
                pltpu.SemaphoreType.DMA((C,)),
                pltpu.SemaphoreType.DMA((C,)),
                pltpu.SemaphoreType.DMA((C,)),
                pltpu.SemaphoreType.DMA((C,)),
                pltpu.SemaphoreType.DMA((C,)),
                pltpu.SemaphoreType.DMA((C,)),
                pltpu.SemaphoreType.DMA((C,)),
            ],
        ),
        out_shape=[
            jax.ShapeDtypeStruct((HALF, N), jnp.float32),
            jax.ShapeDtypeStruct((HB, N), jnp.float32),
        ],
        compiler_params=pltpu.CompilerParams(
            dimension_semantics=("arbitrary", "arbitrary"),
            collective_id=0,
            vmem_limit_bytes=56 * 1024 * 1024,
        ),
    )(s, x, dy)


def kernel(x, dy):
    ix = lax.axis_index("x")
    iy = lax.axis_index("y")
    iz = lax.axis_index("z")
    q = 2 * ix + iz
    h1 = 4 * iy + q
    h2 = 4 * (1 - iy) + q
    idx = []
    for j in range(C):
        idx.append(h2 * C + j)
        idx.append(h1 * C + j)
    s = jnp.stack(idx).astype(jnp.int32)
    out, _ = _fused(s, x, dy)
    return out


# device time: 480195 ns/iter; 3.2869x vs baseline; 1.1673x over previous
import jax
import jax.numpy as jnp
from jax import lax
from jax.experimental import pallas as pl
from jax.experimental.pallas import tpu as pltpu

M = 4096
N = 8192
K = 4096
HALF = 2048
HB = 512

C = 4
CH = HB // C
R = 256
T = 4
BK = 256
NK = K // BK

_MESH = pl.DeviceIdType.MESH


def _fused_body(
    s_ref,
    x_ref,
    dy_ref,
    o_ref,
    yb_ref,
    acc_c,
    acc_o,
    vb_ref,
    l1,
    l2,
    ya_s,
    ya_r,
    xp_s,
    xp_r,
    zp_s,
    zp_r,
    xr_s,
    xr_r,
    zr_s,
    zr_r,
):
    del s_ref
    t = pl.program_id(0)
    k = pl.program_id(1)
    r = t // 2
    is_cross = (t % 2) == 0

    ix = lax.axis_index("x")
    iy = lax.axis_index("y")
    iz = lax.axis_index("z")
    y_nbr = (ix, 1 - iy, iz)
    x_nbr = (1 - ix, iy, iz)
    z_nbr = (ix, iy, 1 - iz)

    q = 2 * ix + iz
    off = q * HB
    off_x = (2 * (1 - ix) + iz) * HB
    off_z = (2 * ix + (1 - iz)) * HB
    off_d = (2 * (1 - ix) + (1 - iz)) * HB

    def y_edge(c):
        return pltpu.make_async_remote_copy(
            src_ref=acc_c.at[pl.ds((c % 2) * CH, CH)],
            dst_ref=yb_ref.at[pl.ds(c * CH, CH)],
            send_sem=ya_s.at[c],
            recv_sem=ya_r.at[c],
            device_id=y_nbr,
            device_id_type=_MESH,
        )

    def prim_edge(c, send, recv, dev):
        return pltpu.make_async_remote_copy(
            src_ref=acc_o.at[c // 2, pl.ds((c % 2) * CH, CH)],
            dst_ref=o_ref.at[pl.ds(off + c * CH, CH)],
            send_sem=send.at[c],
            recv_sem=recv.at[c],
            device_id=dev,
            device_id_type=_MESH,
        )

    def o_edge(row_off, send, recv, c, dev):
        sl = pl.ds(row_off + c * CH, CH)
        return pltpu.make_async_remote_copy(
            src_ref=o_ref.at[sl],
            dst_ref=o_ref.at[sl],
            send_sem=send.at[c],
            recv_sem=recv.at[c],
            device_id=dev,
            device_id_type=_MESH,
        )

    def local_store(c):
        return pltpu.make_async_copy(
            acc_o.at[c // 2, pl.ds((c % 2) * CH, CH)],
            o_ref.at[pl.ds(off + c * CH, CH)],
            l2.at[c],
        )

    @pl.when((t == 0) & (k == 0))
    def _():
        barrier = pltpu.get_barrier_semaphore()
        for nbr in (x_nbr, y_nbr, z_nbr):
            pl.semaphore_signal(
                barrier, inc=1, device_id=nbr, device_id_type=_MESH
            )
        pl.semaphore_wait(barrier, 3)

    prod = lax.dot_general(
        x_ref[...],
        dy_ref[...],
        dimension_numbers=(((0,), (0,)), ((), ())),
        preferred_element_type=jnp.float32,
    )

    @pl.when(is_cross)
    def _():
        @pl.when(k == 0)
        def _():
            @pl.when(r >= 1)
            def _():
                y_edge(0).wait_send()
                y_edge(1).wait_send()

            acc_c[...] = prod

        @pl.when(k != 0)
        def _():
            acc_c[...] += prod

    @pl.when(jnp.logical_not(is_cross))
    def _():
        @pl.when(k == 0)
        def _():
            acc_o[r, :, :] = prod

        @pl.when(k != 0)
        def _():
            acc_o[r, :, :] += prod

    @pl.when((k == NK - 1) & is_cross)
    def _():
        y_edge(2 * r).start()
        y_edge(2 * r + 1).start()

    @pl.when((k == NK - 1) & jnp.logical_not(is_cross))
    def _():
        for cc in range(2):
            c = 2 * r + cc
            y_edge(c).wait_recv()
            cp = pltpu.make_async_copy(
                yb_ref.at[pl.ds(c * CH, CH)], vb_ref, l1
            )
            cp.start()
            cp.wait()
            acc_o[r, pl.ds(cc * CH, CH), :] += vb_ref[...]
            local_store(c).start()
            prim_edge(c, xp_s, xp_r, x_nbr).start()
            prim_edge(c, zp_s, zp_r, z_nbr).start()

    @pl.when((t == T - 1) & (k == NK - 1))
    def _():
        relays = []
        for c in range(C // 2):
            o_edge(off_x, xp_s, xp_r, c, x_nbr).wait_recv()
            rr = o_edge(off_x, zr_s, zr_r, c, z_nbr)
            rr.start()
            relays.append(rr)
        for c in range(C // 2, C):
            o_edge(off_z, zp_s, zp_r, c, z_nbr).wait_recv()
            rr = o_edge(off_z, xr_s, xr_r, c, x_nbr)
            rr.start()
            relays.append(rr)
        for c in range(C // 2, C):
            o_edge(off_x, xp_s, xp_r, c, x_nbr).wait_recv()
        for c in range(C // 2):
            o_edge(off_z, zp_s, zp_r, c, z_nbr).wait_recv()
        for c in range(C // 2):
            o_edge(off_d, zr_s, zr_r, c, z_nbr).wait_recv()
        for c in range(C // 2, C):
            o_edge(off_d, xr_s, xr_r, c, x_nbr).wait_recv()
        for c in range(2, C):
            y_edge(c).wait_send()
        for c in range(C):
            prim_edge(c, xp_s, xp_r, x_nbr).wait_send()
            prim_edge(c, zp_s, zp_r, z_nbr).wait_send()
            local_store(c).wait()
        for rr in relays:
            rr.wait_send()


def _fused(s, x, dy):
    return pl.pallas_call(
        _fused_body,
        grid_spec=pltpu.PrefetchScalarGridSpec(
            num_scalar_prefetch=1,
            grid=(T, NK),
            in_specs=[
                pl.BlockSpec((BK, R), lambda t, k, s: (k, s[t])),
                pl.BlockSpec((BK, N), lambda t, k, s: (k, 0)),
            ],
            out_specs=[
                pl.BlockSpec(memory_space=pl.ANY),
                pl.BlockSpec(memory_space=pl.ANY),
            ],
            scratch_shapes=[
                pltpu.VMEM((R, N), jnp.float32),
                pltpu.VMEM((2, R, N), jnp.float32),
                pltpu.VMEM((CH, N), jnp.float32),
                pltpu.SemaphoreType.DMA,
                pltpu.SemaphoreType.DMA((C,)),
                pltpu.SemaphoreType.DMA((C,)),
                pltpu.SemaphoreType.DMA((C,)),
                pltpu.SemaphoreType.DMA((C,)),
                pltpu.SemaphoreType.DMA((C,)),
                pltpu.SemaphoreType.DMA((C,)),
                pltpu.SemaphoreType.DMA((C,)),
                pltpu.SemaphoreType.DMA((C,)),
                pltpu.SemaphoreType.DMA((C,)),
                pltpu.SemaphoreType.DMA((C,)),
                pltpu.SemaphoreType.DMA((C,)),
            ],
        ),
        out_shape=[
            jax.ShapeDtypeStruct((HALF, N), jnp.float32),
            jax.ShapeDtypeStruct((HB, N), jnp.float32),
        ],
        compiler_params=pltpu.CompilerParams(
            dimension_semantics=("arbitrary", "arbitrary"),
            collective_id=0,
            vmem_limit_bytes=56 * 1024 * 1024,
        ),
    )(s, x, dy)


def kernel(x, dy):
    ix = lax.axis_index("x")
    iy = lax.axis_index("y")
    iz = lax.axis_index("z")
    q = 2 * ix + iz
    h1 = 4 * iy + q
    h2 = 4 * (1 - iy) + q
    s = jnp.stack([2 * h2, 2 * h1, 2 * h2 + 1, 2 * h1 + 1]).astype(jnp.int32)
    out, _ = _fused(s, x, dy)
    return out


# device time: 475845 ns/iter; 3.3169x vs baseline; 1.0091x over previous
import jax
import jax.numpy as jnp
from jax import lax
from jax.experimental import pallas as pl
from jax.experimental.pallas import tpu as pltpu

M = 4096
N = 8192
K = 4096
HALF = 2048
HB = 512

C = 8
CH = HB // C
R = 256
CPS = R // CH
T = 4
BK = 256
NK = K // BK

_MESH = pl.DeviceIdType.MESH


def _fused_body(
    s_ref,
    x_ref,
    dy_ref,
    o_ref,
    yb_ref,
    acc_c,
    acc_o,
    vb_ref,
    l1,
    l2,
    ya_s,
    ya_r,
    xp_s,
    xp_r,
    zp_s,
    zp_r,
    xr_s,
    xr_r,
    zr_s,
    zr_r,
):
    del s_ref
    t = pl.program_id(0)
    k = pl.program_id(1)
    r = t // 2
    is_cross = (t % 2) == 0

    ix = lax.axis_index("x")
    iy = lax.axis_index("y")
    iz = lax.axis_index("z")
    y_nbr = (ix, 1 - iy, iz)
    x_nbr = (1 - ix, iy, iz)
    z_nbr = (ix, iy, 1 - iz)

    q = 2 * ix + iz
    off = q * HB
    off_x = (2 * (1 - ix) + iz) * HB
    off_z = (2 * ix + (1 - iz)) * HB
    off_d = (2 * (1 - ix) + (1 - iz)) * HB

    def y_edge(c):
        return pltpu.make_async_remote_copy(
            src_ref=acc_c.at[pl.ds((c % CPS) * CH, CH)],
            dst_ref=yb_ref.at[pl.ds(c * CH, CH)],
            send_sem=ya_s.at[c],
            recv_sem=ya_r.at[c],
            device_id=y_nbr,
            device_id_type=_MESH,
        )

    def prim_edge(c, send, recv, dev):
        return pltpu.make_async_remote_copy(
            src_ref=acc_o.at[c // CPS, pl.ds((c % CPS) * CH, CH)],
            dst_ref=o_ref.at[pl.ds(off + c * CH, CH)],
            send_sem=send.at[c],
            recv_sem=recv.at[c],
            device_id=dev,
            device_id_type=_MESH,
        )

    def o_edge(row_off, send, recv, c, dev):
        sl = pl.ds(row_off + c * CH, CH)
        return pltpu.make_async_remote_copy(
            src_ref=o_ref.at[sl],
            dst_ref=o_ref.at[sl],
            send_sem=send.at[c],
            recv_sem=recv.at[c],
            device_id=dev,
            device_id_type=_MESH,
        )

    def local_store(c):
        return pltpu.make_async_copy(
            acc_o.at[c // CPS, pl.ds((c % CPS) * CH, CH)],
            o_ref.at[pl.ds(off + c * CH, CH)],
            l2.at[c],
        )

    @pl.when((t == 0) & (k == 0))
    def _():
        barrier = pltpu.get_barrier_semaphore()
        for nbr in (x_nbr, y_nbr, z_nbr):
            pl.semaphore_signal(
                barrier, inc=1, device_id=nbr, device_id_type=_MESH
            )
        pl.semaphore_wait(barrier, 3)

    prod = lax.dot_general(
        x_ref[...],
        dy_ref[...],
        dimension_numbers=(((0,), (0,)), ((), ())),
        preferred_element_type=jnp.float32,
    )

    @pl.when(is_cross)
    def _():
        @pl.when(k == 0)
        def _():
            @pl.when(r >= 1)
            def _():
                for cc in range(CPS):
                    y_edge(cc).wait_send()

            acc_c[...] = prod

        @pl.when(k != 0)
        def _():
            acc_c[...] += prod

    @pl.when(jnp.logical_not(is_cross))
    def _():
        @pl.when(k == 0)
        def _():
            acc_o[r, :, :] = prod

        @pl.when(k != 0)
        def _():
            acc_o[r, :, :] += prod

    @pl.when((k == NK - 1) & is_cross)
    def _():
        for cc in range(CPS):
            y_edge(CPS * r + cc).start()

    @pl.when((k == NK - 1) & jnp.logical_not(is_cross))
    def _():
        for cc in range(CPS):
            c = CPS * r + cc
            y_edge(c).wait_recv()
            cp = pltpu.make_async_copy(
                yb_ref.at[pl.ds(c * CH, CH)], vb_ref, l1
            )
            cp.start()
            cp.wait()
            acc_o[r, pl.ds(cc * CH, CH), :] += vb_ref[...]
            local_store(c).start()
            prim_edge(c, xp_s, xp_r, x_nbr).start()
            prim_edge(c, zp_s, zp_r, z_nbr).start()

    @pl.when((t == T - 1) & (k == NK - 1))
    def _():
        relays = []
        for c in range(C // 2):
            o_edge(off_x, xp_s, xp_r, c, x_nbr).wait_recv()
            rr = o_edge(off_x, zr_s, zr_r, c, z_nbr)
            rr.start()
            relays.append(rr)
        for c in range(C // 2, C):
            o_edge(off_z, zp_s, zp_r, c, z_nbr).wait_recv()
            rr = o_edge(off_z, xr_s, xr_r, c, x_nbr)
            rr.start()
            relays.append(rr)
        for c in range(C // 2, C):
            o_edge(off_x, xp_s, xp_r, c, x_nbr).wait_recv()
        for c in range(C // 2):
            o_edge(off_z, zp_s, zp_r, c, z_nbr).wait_recv()
        for c in range(C // 2):
            o_edge(off_d, zr_s, zr_r, c, z_nbr).wait_recv()
        for c in range(C // 2, C):
            o_edge(off_d, xr_s, xr_r, c, x_nbr).wait_recv()
        for c in range(CPS, C):
            y_edge(c).wait_send()
        for c in range(C):
            prim_edge(c, xp_s, xp_r, x_nbr).wait_send()
            prim_edge(c, zp_s, zp_r, z_nbr).wait_send()
            local_store(c).wait()
        for rr in relays:
            rr.wait_send()


def _fused(s, x, dy):
    return pl.pallas_call(
        _fused_body,
        grid_spec=pltpu.PrefetchScalarGridSpec(
            num_scalar_prefetch=1,
            grid=(T, NK),
            in_specs=[
                pl.BlockSpec((BK, R), lambda t, k, s: (k, s[t])),
                pl.BlockSpec((BK, N), lambda t, k, s: (k, 0)),
            ],
            out_specs=[
                pl.BlockSpec(memory_space=pl.ANY),
                pl.BlockSpec(memory_space=pl.ANY),
            ],
            scratch_shapes=[
                pltpu.VMEM((R, N), jnp.float32),
                pltpu.VMEM((2, R, N), jnp.float32),
                pltpu.VMEM((CH, N), jnp.float32),
                pltpu.SemaphoreType.DMA,
                pltpu.SemaphoreType.DMA((C,)),
                pltpu.SemaphoreType.DMA((C,)),
                pltpu.SemaphoreType.DMA((C,)),
                pltpu.SemaphoreType.DMA((C,)),
                pltpu.SemaphoreType.DMA((C,)),
                pltpu.SemaphoreType.DMA((C,)),
                pltpu.SemaphoreType.DMA((C,)),
                pltpu.SemaphoreType.DMA((C,)),
                pltpu.SemaphoreType.DMA((C,)),
                pltpu.SemaphoreType.DMA((C,)),
                pltpu.SemaphoreType.DMA((C,)),
            ],
        ),
        out_shape=[
            jax.ShapeDtypeStruct((HALF, N), jnp.float32),
            jax.ShapeDtypeStruct((HB, N), jnp.float32),
        ],
        compiler_params=pltpu.CompilerParams(
            dimension_semantics=("arbitrary", "arbitrary"),
            collective_id=0,
            vmem_limit_bytes=56 * 1024 * 1024,
        ),
    )(s, x, dy)


def kernel(x, dy):
    ix = lax.axis_index("x")
    iy = lax.axis_index("y")
    iz = lax.axis_index("z")
    q = 2 * ix + iz
    h1 = 4 * iy + q
    h2 = 4 * (1 - iy) + q
    s = jnp.stack([2 * h2, 2 * h1, 2 * h2 + 1, 2 * h1 + 1]).astype(jnp.int32)
    out, _ = _fused(s, x, dy)
    return out
